# baseline (speedup 1.0000x reference)
import functools

import jax
import jax.numpy as jnp
from jax import lax
from jax.experimental import pallas as pl
from jax.experimental.pallas import tpu as pltpu
from jax.experimental.pallas import tpu_sc as plsc

B = 1024
P = 10
TAIL = 11
LOUT = P + TAIL
D = 1024
SL, LN = 8, 128

NC = 2
NS = 16
NW = NC * NS
BPW = B // NW

_mesh = plsc.VectorSubcoreMesh(core_axis_name="c", subcore_axis_name="s")


@functools.partial(
    pl.kernel,
    out_type=jax.ShapeDtypeStruct((B, LOUT, SL, LN), jnp.float32),
    mesh=_mesh,
    scratch_types=[
        pltpu.VMEM((1, TAIL, SL, LN), jnp.float32),
        pltpu.SemaphoreType.DMA,
        pltpu.SemaphoreType.DMA,
        pltpu.SemaphoreType.DMA,
    ],
)
def _sc_fill(vtok_hbm, end_hbm, rep_hbm, out_hbm, tail_v, sem_v, sem_f, sem_t):
    wid = lax.axis_index("s") * NC + lax.axis_index("c")
    base = wid * BPW

    vtok_cp = pltpu.async_copy(
        vtok_hbm.at[pl.ds(base, BPW)],
        out_hbm.at[pl.ds(base, BPW), pl.ds(0, P)],
        sem_v,
    )

    fill_cps = [pltpu.async_copy(end_hbm, tail_v.at[:, pl.ds(0, 1)], sem_f)]
    for j in range(P):
        fill_cps.append(
            pltpu.async_copy(rep_hbm, tail_v.at[:, pl.ds(1 + j, 1)], sem_f))
    for cp in fill_cps:
        cp.wait()

    tail_cps = []
    for b in range(BPW):
        tail_cps.append(pltpu.async_copy(
            tail_v,
            out_hbm.at[pl.ds(base + b, 1), pl.ds(P, TAIL)],
            sem_t,
        ))
    for cp in tail_cps:
        cp.wait()
    vtok_cp.wait()


def kernel(categories, vtok, end, zero):
    rep = jnp.where(categories[0, 11] == 0, zero, end)
    out4 = _sc_fill(
        vtok.reshape(B, P, SL, LN),
        end.reshape(1, 1, SL, LN),
        rep.reshape(1, 1, SL, LN),
    )
    return out4.reshape(B, LOUT, D)


# speedup vs baseline: 8.6043x; 8.6043x over previous
import jax
import jax.numpy as jnp
from jax.experimental import pallas as pl

B = 1024
P = 10
TAIL = 11
LOUT = P + TAIL
D = 1024

BB = 64


def _fill_body(vtok_ref, end_ref, rep_ref, out_ref):
    bb = vtok_ref.shape[0]
    end_row = end_ref[...][None, :, :]
    rep_row = rep_ref[...][None, :, :]
    tail = jnp.concatenate(
        [jnp.broadcast_to(end_row, (bb, 1, D)),
         jnp.broadcast_to(rep_row, (bb, P, D))], axis=1)
    out_ref[...] = jnp.concatenate([vtok_ref[...], tail], axis=1)


def kernel(categories, vtok, end, zero):
    rep = jnp.where(categories[0, 11] == 0, zero, end)
    return pl.pallas_call(
        _fill_body,
        grid=(B // BB,),
        in_specs=[
            pl.BlockSpec((BB, P, D), lambda i: (i, 0, 0)),
            pl.BlockSpec((1, D), lambda i: (0, 0)),
            pl.BlockSpec((1, D), lambda i: (0, 0)),
        ],
        out_specs=pl.BlockSpec((BB, LOUT, D), lambda i: (i, 0, 0)),
        out_shape=jax.ShapeDtypeStruct((B, LOUT, D), jnp.float32),
    )(vtok, end, rep)


# speedup vs baseline: 8.7303x; 1.0146x over previous
import jax
import jax.numpy as jnp
from jax.experimental import pallas as pl

B = 1024
P = 10
TAIL = 11
LOUT = P + TAIL
D = 1024

BB = 128


def _fill_body(vtok_ref, end_ref, rep_ref, out_ref):
    bb = vtok_ref.shape[0]
    end_row = end_ref[...][None, :, :]
    rep_row = rep_ref[...][None, :, :]
    tail = jnp.concatenate(
        [jnp.broadcast_to(end_row, (bb, 1, D)),
         jnp.broadcast_to(rep_row, (bb, P, D))], axis=1)
    out_ref[...] = jnp.concatenate([vtok_ref[...], tail], axis=1)


def kernel(categories, vtok, end, zero):
    rep = jnp.where(categories[0, 11] == 0, zero, end)
    return pl.pallas_call(
        _fill_body,
        grid=(B // BB,),
        in_specs=[
            pl.BlockSpec((BB, P, D), lambda i: (i, 0, 0)),
            pl.BlockSpec((1, D), lambda i: (0, 0)),
            pl.BlockSpec((1, D), lambda i: (0, 0)),
        ],
        out_specs=pl.BlockSpec((BB, LOUT, D), lambda i: (i, 0, 0)),
        out_shape=jax.ShapeDtypeStruct((B, LOUT, D), jnp.float32),
    )(vtok, end, rep)
